# baseline (speedup 1.0000x reference)
import jax
import jax.numpy as jnp
from jax.experimental import pallas as pl

_ROWS_PER_BLOCK = 512


def _select_body(m_ref, t_ref, x_ref, o_ref):
    i = pl.program_id(0)
    m = m_ref[i, :]
    tok = t_ref[0, :]
    o_ref[:, :] = jnp.where(m[:, None] > 0.5, tok[None, :], x_ref[:, :])


def kernel(x, patch_mask, mask_token):
    N, L, H, W, D = x.shape
    rows = N * L * H * W
    xf = x.reshape(rows, D)
    nblk = rows // _ROWS_PER_BLOCK
    mf = patch_mask.reshape(nblk, _ROWS_PER_BLOCK).astype(jnp.float32)

    out = pl.pallas_call(
        _select_body,
        grid=(nblk,),
        in_specs=[
            pl.BlockSpec((nblk, _ROWS_PER_BLOCK), lambda i: (0, 0)),
            pl.BlockSpec((1, D), lambda i: (0, 0)),
            pl.BlockSpec((_ROWS_PER_BLOCK, D), lambda i: (i, 0)),
        ],
        out_specs=pl.BlockSpec((_ROWS_PER_BLOCK, D), lambda i: (i, 0)),
        out_shape=jax.ShapeDtypeStruct((rows, D), x.dtype),
    )(mf, mask_token, xf)

    return (out.reshape(x.shape), patch_mask)


# speedup vs baseline: 1.2005x; 1.2005x over previous
import jax
import jax.numpy as jnp
from jax.experimental import pallas as pl
from jax.experimental.pallas import tpu as pltpu

_ROWS_PER_BLOCK = 1024


def _select_body(m_ref, t_ref, x_ref, o_ref):
    i = pl.program_id(0)
    m = m_ref[i, :]
    tok = t_ref[0, :]
    o_ref[:, :] = jnp.where(m[:, None] > 0.5, tok[None, :], x_ref[:, :])


def kernel(x, patch_mask, mask_token):
    N, L, H, W, D = x.shape
    rows = N * L * H * W
    xf = x.reshape(rows, D)
    nblk = rows // _ROWS_PER_BLOCK
    mf = patch_mask.reshape(nblk, _ROWS_PER_BLOCK).astype(jnp.float32)

    out = pl.pallas_call(
        _select_body,
        grid=(nblk,),
        in_specs=[
            pl.BlockSpec((nblk, _ROWS_PER_BLOCK), lambda i: (0, 0)),
            pl.BlockSpec((1, D), lambda i: (0, 0)),
            pl.BlockSpec((_ROWS_PER_BLOCK, D), lambda i: (i, 0)),
        ],
        out_specs=pl.BlockSpec((_ROWS_PER_BLOCK, D), lambda i: (i, 0)),
        out_shape=jax.ShapeDtypeStruct((rows, D), x.dtype),
        compiler_params=pltpu.CompilerParams(
            dimension_semantics=("parallel",),
        ),
    )(mf, mask_token, xf)

    return (out.reshape(x.shape), patch_mask)


# speedup vs baseline: 1.2711x; 1.0588x over previous
import jax
import jax.numpy as jnp
from jax.experimental import pallas as pl
from jax.experimental.pallas import tpu as pltpu

_ROWS_PER_BLOCK = 4608


def _select_body(m_ref, t_ref, x_ref, o_ref):
    i = pl.program_id(0)
    m = m_ref[i, :]
    tok = t_ref[0, :]
    o_ref[:, :] = jnp.where(m[:, None] > 0.5, tok[None, :], x_ref[:, :])


def kernel(x, patch_mask, mask_token):
    N, L, H, W, D = x.shape
    rows = N * L * H * W
    xf = x.reshape(rows, D)
    nblk = rows // _ROWS_PER_BLOCK
    mf = patch_mask.reshape(nblk, _ROWS_PER_BLOCK).astype(jnp.float32)

    out = pl.pallas_call(
        _select_body,
        grid=(nblk,),
        in_specs=[
            pl.BlockSpec((nblk, _ROWS_PER_BLOCK), lambda i: (0, 0)),
            pl.BlockSpec((1, D), lambda i: (0, 0)),
            pl.BlockSpec((_ROWS_PER_BLOCK, D), lambda i: (i, 0)),
        ],
        out_specs=pl.BlockSpec((_ROWS_PER_BLOCK, D), lambda i: (i, 0)),
        out_shape=jax.ShapeDtypeStruct((rows, D), x.dtype),
        compiler_params=pltpu.CompilerParams(
            dimension_semantics=("parallel",),
        ),
    )(mf, mask_token, xf)

    return (out.reshape(x.shape), patch_mask)
